# Initial kernel scaffold; baseline (speedup 1.0000x reference)
import jax
import jax.numpy as jnp
from jax.experimental import pallas as pl


def kernel(x, edge, e_attr, bbox_idx, bbox, stat_feats, Wm, We, bm, Wu, bu, Ws, Ws2, bs, Wf, bf, Wfs, bfs, Wp1, bp1, Wp2, bp2, Wp3, bp3):
    raise NotImplementedError("write your pallas kernel here")


# speedup vs baseline: 1.0233x; 1.0233x over previous
import jax
import jax.numpy as jnp
from jax.experimental import pallas as pl

_N = 10000
_NB = 500
_NBLK = 4


def _head(out_cls, Wp1, bp1, Wp2, bp2, Wp3, bp3):
    def body(oc, w1, b1, w2, b2, w3, b3, out):
        h = jnp.maximum(oc[...] @ w1[...] + b1[...], 0.0)
        h = jnp.maximum(h @ w2[...] + b2[...], 0.0)
        out[...] = h[...] @ w3[...] + b3[...]

    return pl.pallas_call(
        body,
        out_shape=jax.ShapeDtypeStruct((out_cls.shape[0], 21), jnp.float32),
    )(out_cls, Wp1, bp1.reshape(1, -1), Wp2, bp2.reshape(1, -1), Wp3,
      bp3.reshape(1, -1))


def kernel(x, edge, e_attr, bbox_idx, bbox, stat_feats, Wm, We, bm, Wu, bu,
           Ws, Ws2, bs, Wf, bf, Wfs, bfs, Wp1, bp1, Wp2, bp2, Wp3, bp3):
    src = edge[:, 0]
    dst = edge[:, 1]
    E = src.shape[0]

    def conv(xin, xnode, i):
        msg = jax.nn.relu((xin @ Wm[i])[src] + e_attr @ We[i] + bm[i])
        agg = jax.ops.segment_sum(msg, dst, num_segments=_N)
        deg = jax.ops.segment_sum(jnp.ones((E, 1), jnp.float32), dst,
                                  num_segments=_N)
        agg = agg / jnp.clip(deg, 1.0, None)
        f = jax.nn.relu(xin @ Wu[i, :128] + agg @ Wu[i, 128:] + bu[i])
        fs = jax.nn.relu(xnode @ Ws[i] + agg @ Ws2[i] + bs[i])
        return f, fs

    f, fs = conv(x, x, 0)
    feats = [f]
    feats_super = [fs]
    for i in range(1, _NBLK):
        nf, nfs = conv(feats[-1], feats_super[-1], i)
        feats.append(feats[-1] + nf)
        feats_super.append(feats_super[-1] + nfs)
    feats_c = jnp.concatenate(feats, axis=1)
    fusion = jax.nn.relu(feats_c @ Wf + bf)
    out_feat = jnp.concatenate([fusion, feats_c], axis=1)
    feats_s = jnp.concatenate(feats_super, axis=1)
    s = jax.ops.segment_sum(feats_s, bbox_idx, num_segments=_NB)
    c = jax.ops.segment_sum(jnp.ones((_N, 1), jnp.float32), bbox_idx,
                            num_segments=_NB)
    feats_s = s / jnp.clip(c, 1.0, None)
    fusion_s = jax.nn.relu(feats_s @ Wfs + bfs)
    m = jax.ops.segment_max(out_feat, bbox_idx, num_segments=_NB)
    m = jnp.where(jnp.isfinite(m), m, 0.0)
    out_cls = jnp.concatenate([m, fusion_s, feats_s], axis=1)
    pred = _head(out_cls, Wp1, bp1, Wp2, bp2, Wp3, bp3)
    return (pred, bbox)


# speedup vs baseline: 2.2410x; 2.1900x over previous
import functools

import jax
import jax.numpy as jnp
from jax import lax
from jax.experimental import pallas as pl
from jax.experimental.pallas import tpu as pltpu
from jax.experimental.pallas import tpu_sc as plsc

_N = 10000
_E = 320000
_D = 128
_NB = 500
_NBLK = 4
_NW = 32
_C = 128
_ECH = _E // _C
_ET = (_ECH + _NW - 1) // _NW

_f32 = jnp.float32
_SC_POOL = False


def _mesh():
    return plsc.VectorSubcoreMesh(core_axis_name="c", subcore_axis_name="s")


def _zero_rows(buf, nrows, ncolchunks):
    def body(i, _):
        for cc in range(ncolchunks):
            buf[i, pl.ds(cc * 16, 16)] = jnp.zeros((16,), _f32)
        return 0
    lax.fori_loop(0, nrows, body, 0)


_NP = 10240


def _sc_edge(y, src, dst, ea_flat, we):
    @functools.partial(
        pl.kernel,
        out_type=(jax.ShapeDtypeStruct((_NP, _D), _f32),
                  jax.ShapeDtypeStruct((_NP, _D), _f32)),
        mesh=_mesh(),
        scratch_types=[
            pltpu.VMEM((_C,), jnp.int32),
            pltpu.VMEM((_C,), jnp.int32),
            pltpu.VMEM((_C * 4 + 16,), _f32),
            pltpu.VMEM((_C, _D), _f32),
            pltpu.VMEM((4, _D), _f32),
            pltpu.VMEM((160, _D), _f32),
            pltpu.VMEM_SHARED((_NP, _D), _f32),
            pltpu.SemaphoreType.DMA,
        ],
    )
    def k(y_h, src_h, dst_h, ea_h, we_h, out0_h, out1_h,
          sidx, didx, eav, rows, wev, zbuf, acc, sem):
        c = lax.axis_index("c")
        s = lax.axis_index("s")
        w = s * 2 + c

        _zero_rows(zbuf, 160, _D // 16)
        for kk in range(4):
            pltpu.sync_copy(zbuf, acc.at[pl.ds(s * 640 + kk * 160, 160), :])
        pltpu.sync_copy(we_h, wev)
        wv = [[wev[kk, pl.ds(cc * 16, 16)] for cc in range(_D // 16)]
              for kk in range(4)]
        plsc.subcore_barrier()

        def chunk(t, _):
            g = w + t * _NW

            @pl.when(g < _ECH)
            def _():
                base = g * _C
                pltpu.sync_copy(src_h.at[pl.ds(base, _C)], sidx)
                pltpu.sync_copy(dst_h.at[pl.ds(base, _C)], didx)
                pltpu.sync_copy(ea_h.at[pl.ds(base * 4, _C * 4)],
                                eav.at[pl.ds(0, _C * 4)])
                pltpu.async_copy(y_h.at[sidx], rows, sem).wait()

                def ebody(j, _):
                    av = eav[pl.ds(j * 4, 16)]
                    a0 = av[0]
                    a1 = av[1]
                    a2 = av[2]
                    a3 = av[3]
                    for cc in range(_D // 16):
                        v = rows[j, pl.ds(cc * 16, 16)]
                        v = v + a0 * wv[0][cc] + a1 * wv[1][cc]
                        v = v + a2 * wv[2][cc] + a3 * wv[3][cc]
                        rows[j, pl.ds(cc * 16, 16)] = jnp.maximum(v, 0.0)
                    return 0

                lax.fori_loop(0, _C, ebody, 0)
                pltpu.sync_copy(rows, acc.at[didx], add=True)
            return 0

        lax.fori_loop(0, _ET, chunk, 0)
        plsc.subcore_barrier()

        for kk in range(4):
            pltpu.sync_copy(acc.at[pl.ds(s * 640 + kk * 160, 160), :], zbuf)

            @pl.when(c == 0)
            def _():
                pltpu.sync_copy(
                    zbuf, out0_h.at[pl.ds(s * 640 + kk * 160, 160), :])

            @pl.when(c == 1)
            def _():
                pltpu.sync_copy(
                    zbuf, out1_h.at[pl.ds(s * 640 + kk * 160, 160), :])

    a0, a1 = k(y, src, dst, ea_flat, we)
    return a0[:_N], a1[:_N]


def _sc_pool_all(fusion, feats_c, feats_s, bidx):
    rch = 200
    nswp = (_N // 2) // rch

    def _sweep(src_h, bidx_h, ib, buf, accv, lc, half, is_max):
        init = float("-inf") if is_max else 0.0
        col = pl.multiple_of(lc * 128, 128)
        r0 = half * (_N // 2)

        def ibody(b, _):
            for cc in range(8):
                accv[b, pl.ds(cc * 16, 16)] = jnp.full((16,), init, _f32)
            return 0
        lax.fori_loop(0, _NB, ibody, 0)

        def rsweep(rc, _):
            base = r0 + rc * rch
            pltpu.sync_copy(bidx_h.at[pl.ds(base, rch)],
                            ib.at[pl.ds(0, rch)])
            pltpu.sync_copy(src_h.at[pl.ds(base, rch), pl.ds(col, 128)], buf)

            def rbody(r, _):
                seg = ib[pl.ds(r, 16)][0]
                for cc in range(8):
                    cur = accv[seg, pl.ds(cc * 16, 16)]
                    val = buf[r, pl.ds(cc * 16, 16)]
                    if is_max:
                        accv[seg, pl.ds(cc * 16, 16)] = jnp.maximum(cur, val)
                    else:
                        accv[seg, pl.ds(cc * 16, 16)] = cur + val
                return 0

            lax.fori_loop(0, rch, rbody, 0)
            return 0

        lax.fori_loop(0, nswp, rsweep, 0)

    def _combine_write(out_h, buf, accv, stage, slot, lc, is_max):
        p0 = 0
        for sz in (200, 200, 100):
            pltpu.sync_copy(stage.at[slot, pl.ds(p0, sz), :],
                            buf.at[pl.ds(0, sz), :])

            def cbody(r, _):
                for cc in range(8):
                    cur = accv[p0 + r, pl.ds(cc * 16, 16)]
                    val = buf[r, pl.ds(cc * 16, 16)]
                    if is_max:
                        accv[p0 + r, pl.ds(cc * 16, 16)] = \
                            jnp.maximum(cur, val)
                    else:
                        accv[p0 + r, pl.ds(cc * 16, 16)] = cur + val
                return 0

            lax.fori_loop(0, sz, cbody, 0)
            p0 += sz
        col = pl.multiple_of(lc * 128, 128)
        pltpu.sync_copy(accv, out_h.at[:, pl.ds(col, 128)])

    @functools.partial(
        pl.kernel,
        out_type=(jax.ShapeDtypeStruct((_NB, 1024), _f32),
                  jax.ShapeDtypeStruct((_NB, 512), _f32),
                  jax.ShapeDtypeStruct((_NB, 512), _f32)),
        mesh=_mesh(),
        scratch_types=[
            pltpu.VMEM((rch + 16,), jnp.int32),
            pltpu.VMEM((rch, 128), _f32),
            pltpu.VMEM((_NB, 128), _f32),
            pltpu.VMEM_SHARED((8, _NB, 128), _f32),
        ],
    )
    def k(fus_h, fc_h, fs_h, bidx_h, ofus_h, ofc_h, ofs_h,
          ib, buf, accv, stage):
        c = lax.axis_index("c")
        s = lax.axis_index("s")
        half = s // 8
        slot = s % 8

        @pl.when(c == 0)
        def _():
            _sweep(fus_h, bidx_h, ib, buf, accv, slot, half, True)

        @pl.when((c == 1) & (slot < 4))
        def _():
            _sweep(fc_h, bidx_h, ib, buf, accv, slot, half, True)

        @pl.when((c == 1) & (slot >= 4))
        def _():
            _sweep(fs_h, bidx_h, ib, buf, accv, slot - 4, half, False)

        @pl.when(half == 1)
        def _():
            pltpu.sync_copy(accv, stage.at[slot, :, :])

        plsc.subcore_barrier()

        @pl.when((half == 0) & (c == 0))
        def _():
            _combine_write(ofus_h, buf, accv, stage, slot, slot, True)

        @pl.when((half == 0) & (c == 1) & (slot < 4))
        def _():
            _combine_write(ofc_h, buf, accv, stage, slot, slot, True)

        @pl.when((half == 0) & (c == 1) & (slot >= 4))
        def _():
            _combine_write(ofs_h, buf, accv, stage, slot, slot - 4, False)

    return k(fusion, feats_c, feats_s, bidx)


def _tc_pre(x, xs, wm, wut, wsn, bm_, bu_, bs_):
    rb = 1000

    def body(xr, xsr, wmr, wutr, wsr, bmr, bur, bsr, y, p1, p2):
        xv = xr[...]
        y[...] = xv @ wmr[...] + bmr[...]
        p1[...] = xv @ wutr[...] + bur[...]
        p2[...] = xsr[...] @ wsr[...] + bsr[...]

    row = pl.BlockSpec((rb, _D), lambda i: (i, 0))
    wsp = pl.BlockSpec((_D, _D), lambda i: (0, 0))
    bsp = pl.BlockSpec((1, _D), lambda i: (0, 0))
    o = jax.ShapeDtypeStruct((_N, _D), _f32)
    return pl.pallas_call(
        body,
        grid=(_N // rb,),
        in_specs=[row, row, wsp, wsp, wsp, bsp, bsp, bsp],
        out_specs=[row, row, row],
        out_shape=[o, o, o],
    )(x, xs, wm, wut, wsn, bm_.reshape(1, -1), bu_.reshape(1, -1),
      bs_.reshape(1, -1))


def _tc_post(agg0, agg1, deg, p1, p2, wub, ws2, xres, sres, residual):
    rb = 1000

    def body(*refs):
        if residual:
            (a0, a1, d0, p1r, p2r, wubr, ws2r, xr, sr, xo, so) = refs
        else:
            (a0, a1, d0, p1r, p2r, wubr, ws2r, xo, so) = refs
        aggm = (a0[...] + a1[...]) / jnp.clip(d0[...], 1.0, None)
        f = jnp.maximum(p1r[...] + aggm @ wubr[...], 0.0)
        fs = jnp.maximum(p2r[...] + aggm @ ws2r[...], 0.0)
        if residual:
            xo[...] = xr[...] + f
            so[...] = sr[...] + fs
        else:
            xo[...] = f
            so[...] = fs

    row = pl.BlockSpec((rb, _D), lambda i: (i, 0))
    drow = pl.BlockSpec((rb, 1), lambda i: (i, 0))
    wsp = pl.BlockSpec((_D, _D), lambda i: (0, 0))
    o = jax.ShapeDtypeStruct((_N, _D), _f32)
    ins = [row, row, drow, row, row, wsp, wsp]
    args = [agg0, agg1, deg, p1, p2, wub, ws2]
    if residual:
        ins += [row, row]
        args += [xres, sres]
    return pl.pallas_call(
        body,
        grid=(_N // rb,),
        in_specs=ins,
        out_specs=[row, row],
        out_shape=[o, o],
    )(*args)


def _tc_fusion(feats_c, wf, bf_):
    rb = 1000

    def body(xr, wr, br, out):
        out[...] = jnp.maximum(xr[...] @ wr[...] + br[...], 0.0)

    return pl.pallas_call(
        body,
        grid=(_N // rb,),
        in_specs=[pl.BlockSpec((rb, 512), lambda i: (i, 0)),
                  pl.BlockSpec((512, 1024), lambda i: (0, 0)),
                  pl.BlockSpec((1, 1024), lambda i: (0, 0))],
        out_specs=pl.BlockSpec((rb, 1024), lambda i: (i, 0)),
        out_shape=jax.ShapeDtypeStruct((_N, 1024), _f32),
    )(feats_c, wf, bf_.reshape(1, -1))


def _tc_head(mx_fus, mx_fc, sm_s, cnt_, wfs, bfs_, w1a, w1b, w1c, w1d,
             bp1_, wp2, bp2_, wp3, bp3_):
    def body(mfr, mcr, smr, c0r, wfsr, bfsr, w1ar, w1br, w1cr, w1dr,
             b1r, w2r, b2r, w3r, b3r, out):
        cnt = c0r[...]
        feats_s = smr[...] / jnp.clip(cnt, 1.0, None)
        fus_s = jnp.maximum(feats_s @ wfsr[...] + bfsr[...], 0.0)
        mask = cnt > 0.0
        mfus = jnp.where(mask, mfr[...], 0.0)
        mfc = jnp.where(mask, mcr[...], 0.0)
        h = mfus @ w1ar[...] + mfc @ w1br[...]
        h = h + fus_s @ w1cr[...] + feats_s @ w1dr[...] + b1r[...]
        h = jnp.maximum(h, 0.0)
        h = jnp.maximum(h @ w2r[...] + b2r[...], 0.0)
        out[...] = h @ w3r[...] + b3r[...]

    return pl.pallas_call(
        body,
        out_shape=jax.ShapeDtypeStruct((_NB, 21), _f32),
    )(mx_fus, mx_fc, sm_s, cnt_, wfs, bfs_.reshape(1, -1), w1a, w1b, w1c,
      w1d, bp1_.reshape(1, -1), wp2, bp2_.reshape(1, -1), wp3,
      bp3_.reshape(1, -1))


def kernel(x, edge, e_attr, bbox_idx, bbox, stat_feats, Wm, We, bm, Wu,
                 bu, Ws, Ws2, bs, Wf, bf, Wfs, bfs, Wp1, bp1, Wp2, bp2, Wp3,
                 bp3):
    src = edge[:, 0]
    dst = edge[:, 1]

    deg = jax.ops.segment_sum(jnp.ones((_E, 1), _f32), dst,
                              num_segments=_N)
    cnt = jax.ops.segment_sum(jnp.ones((_N, 1), _f32), bbox_idx,
                              num_segments=_NB)

    X = x
    S = x
    feats = []
    feats_super = []
    for i in range(_NBLK):
        y, p1, p2 = _tc_pre(X, S, Wm[i], Wu[i, :_D], Ws[i], bm[i], bu[i],
                            bs[i])
        a0, a1 = _sc_edge(y, src, dst, e_attr.reshape(-1), We[i])
        X, S = _tc_post(a0, a1, deg, p1, p2, Wu[i, _D:], Ws2[i],
                        X, S, residual=(i > 0))
        feats.append(X)
        feats_super.append(S)

    feats_c = jnp.concatenate(feats, axis=1)
    fusion = _tc_fusion(feats_c, Wf, bf)
    feats_sc = jnp.concatenate(feats_super, axis=1)

    if _SC_POOL:
        mx_fus, mx_fc, sm_s = _sc_pool_all(fusion, feats_c, feats_sc,
                                           bbox_idx)
    else:
        mx_fus = jax.ops.segment_max(fusion, bbox_idx, num_segments=_NB)
        mx_fc = jax.ops.segment_max(feats_c, bbox_idx, num_segments=_NB)
        sm_s = jax.ops.segment_sum(feats_sc, bbox_idx, num_segments=_NB)

    pred = _tc_head(mx_fus, mx_fc, sm_s, cnt, Wfs, bfs,
                    Wp1[:1024], Wp1[1024:1536], Wp1[1536:2560], Wp1[2560:],
                    bp1, Wp2, bp2, Wp3, bp3)
    return (pred, bbox)
